# initial kernel scaffold (unmeasured)
import jax
import jax.numpy as jnp
from jax import lax
from jax.experimental import pallas as pl
from jax.experimental.pallas import tpu as pltpu

N_Z = 4


def kernel(x, W):
    t, d = x.shape
    _, v_loc = W.shape
    v_glob = N_Z * v_loc

    def body(x_ref, w_ref, out_ref, acc_ref, comm_ref, send_sems, recv_sems):
        my_x = lax.axis_index("x")
        my_y = lax.axis_index("y")
        my_z = lax.axis_index("z")
        left_z = (my_z + N_Z - 1) % N_Z
        right_z = (my_z + 1) % N_Z

        barrier_sem = pltpu.get_barrier_semaphore()
        for nbr_z in [left_z, right_z]:
            pl.semaphore_signal(
                barrier_sem, inc=1,
                device_id=(my_x, my_y, nbr_z),
                device_id_type=pl.DeviceIdType.MESH,
            )
        pl.semaphore_wait(barrier_sem, 2)

        logits = jnp.dot(
            x_ref[:].astype(jnp.bfloat16),
            w_ref[:].astype(jnp.bfloat16),
            preferred_element_type=jnp.float32,
        ).astype(jnp.bfloat16)
        comm_ref[0] = logits
        acc_ref[:, pl.ds(my_z * v_loc, v_loc)] = logits

        for h in range(N_Z - 1):
            send_slot = h % 2
            recv_slot = (h + 1) % 2
            rdma = pltpu.make_async_remote_copy(
                src_ref=comm_ref.at[send_slot],
                dst_ref=comm_ref.at[recv_slot],
                send_sem=send_sems.at[send_slot],
                recv_sem=recv_sems.at[recv_slot],
                device_id=(my_x, my_y, right_z),
                device_id_type=pl.DeviceIdType.MESH,
            )
            rdma.start()
            rdma.wait()
            origin = (my_z + N_Z - h - 1) % N_Z
            acc_ref[:, pl.ds(origin * v_loc, v_loc)] = comm_ref[recv_slot]

        g = acc_ref[:].astype(jnp.float32)
        m = jnp.max(g, axis=-1, keepdims=True)
        e = jnp.exp(g - m)
        s = jnp.sum(e, axis=-1, keepdims=True)
        out_ref[:] = e / s

    return pl.pallas_call(
        body,
        out_shape=jax.ShapeDtypeStruct((t, v_glob), jnp.float32),
        in_specs=[
            pl.BlockSpec(memory_space=pltpu.VMEM),
            pl.BlockSpec(memory_space=pltpu.VMEM),
        ],
        out_specs=pl.BlockSpec(memory_space=pltpu.VMEM),
        scratch_shapes=[
            pltpu.VMEM((t, v_glob), jnp.bfloat16),
            pltpu.VMEM((2, t, v_loc), jnp.bfloat16),
            pltpu.SemaphoreType.DMA((2,)),
            pltpu.SemaphoreType.DMA((2,)),
        ],
        compiler_params=pltpu.CompilerParams(collective_id=0),
    )(x, W)


# baseline (device time: 105056 ns/iter reference)
import jax
import jax.numpy as jnp
from jax import lax
from jax.experimental import pallas as pl
from jax.experimental.pallas import tpu as pltpu

N_Z = 4


def kernel(x, W):
    t, d = x.shape
    _, v_loc = W.shape
    v_glob = N_Z * v_loc

    def body(x_ref, w_ref, out_ref, acc_ref, comm_ref, send_sems, recv_sems):
        my_x = lax.axis_index("x")
        my_y = lax.axis_index("y")
        my_z = lax.axis_index("z")
        left_z = (my_z + N_Z - 1) % N_Z
        right_z = (my_z + 1) % N_Z

        barrier_sem = pltpu.get_barrier_semaphore()
        for nbr_z in [left_z, right_z]:
            pl.semaphore_signal(
                barrier_sem, inc=1,
                device_id=(my_x, my_y, nbr_z),
                device_id_type=pl.DeviceIdType.MESH,
            )
        pl.semaphore_wait(barrier_sem, 2)

        logits = jnp.dot(
            x_ref[:].astype(jnp.bfloat16),
            w_ref[:].astype(jnp.bfloat16),
            preferred_element_type=jnp.float32,
        ).astype(jnp.bfloat16)
        comm_ref[0] = logits
        acc_ref[:, pl.ds(my_z * v_loc, v_loc)] = logits

        for h in range(N_Z - 1):
            send_slot = h % 2
            recv_slot = (h + 1) % 2
            rdma = pltpu.make_async_remote_copy(
                src_ref=comm_ref.at[send_slot],
                dst_ref=comm_ref.at[recv_slot],
                send_sem=send_sems.at[send_slot],
                recv_sem=recv_sems.at[recv_slot],
                device_id=(my_x, my_y, right_z),
                device_id_type=pl.DeviceIdType.MESH,
            )
            rdma.start()
            rdma.wait()
            origin = (my_z + N_Z - h - 1) % N_Z
            acc_ref[:, pl.ds(origin * v_loc, v_loc)] = comm_ref[recv_slot]

        g = acc_ref[:].astype(jnp.float32)
        m = jnp.max(g, axis=-1, keepdims=True)
        e = jnp.exp(g - m)
        s = jnp.sum(e, axis=-1, keepdims=True)
        out_ref[:] = e / s

    return pl.pallas_call(
        body,
        out_shape=jax.ShapeDtypeStruct((t, v_glob), jnp.float32),
        in_specs=[
            pl.BlockSpec(memory_space=pltpu.VMEM),
            pl.BlockSpec(memory_space=pltpu.VMEM),
        ],
        out_specs=pl.BlockSpec(memory_space=pltpu.VMEM),
        scratch_shapes=[
            pltpu.VMEM((t, v_glob), jnp.bfloat16),
            pltpu.VMEM((2, t, v_loc), jnp.bfloat16),
            pltpu.SemaphoreType.DMA((2,)),
            pltpu.SemaphoreType.DMA((2,)),
        ],
        compiler_params=pltpu.CompilerParams(
            collective_id=0,
            vmem_limit_bytes=100 * 1024 * 1024,
        ),
    )(x, W)
